# baseline (speedup 1.0000x reference)
import functools

import jax
import jax.numpy as jnp
from jax import lax
from jax.experimental import pallas as pl
from jax.experimental.pallas import tpu as pltpu
from jax.experimental.pallas import tpu_sc as plsc

_B, _H, _D, _LV = 32, 16, 64, 64
_T = 65536
_L = _T // _B
_CH = 32
_NCH = _L // _CH
_SCALE = 0.125
_LN2 = 0.6931471805599453

_mesh = plsc.VectorSubcoreMesh(core_axis_name="c", subcore_axis_name="s")
_params = pltpu.CompilerParams(use_tc_tiling_on_sc=False,
                               needs_layout_passes=False,
                               disable_bounds_checks=True)


@functools.partial(
    pl.kernel,
    out_type=(
        jax.ShapeDtypeStruct((_B, _L, _H), jnp.float32),
        jax.ShapeDtypeStruct((_B, _H), jnp.float32),
    ),
    mesh=_mesh,
    scratch_types=(
        pltpu.VMEM((_NCH, _CH), jnp.int32),
        pltpu.VMEM((_D, _H), jnp.float32),
        pltpu.VMEM((_CH, 8, 128), jnp.float32),
        pltpu.VMEM((_CH, 8, 128), jnp.float32),
        pltpu.VMEM((_L, _H), jnp.float32),
        pltpu.VMEM((_H,), jnp.float32),
        pltpu.SemaphoreType.DMA,
        pltpu.SemaphoreType.DMA,
    ),
    compiler_params=_params,
)
def _sc_qk(qr_hbm, k_hbm, idx_hbm, lg_hbm, m_hbm,
           idx_v, q_r, kb0, kb1, logits, mbuf, sem0, sem1):
    b = lax.axis_index("s") * 2 + lax.axis_index("c")

    pltpu.sync_copy(idx_hbm.at[b], idx_v)
    pltpu.sync_copy(qr_hbm.at[b], q_r)

    hlane = lax.broadcasted_iota(jnp.int32, (16,), 0)
    base64 = hlane * _D

    def fire(c, kb, sem):
        @pl.when(c < _NCH)
        def _():
            pltpu.make_async_copy(k_hbm.at[idx_v.at[c]], kb, sem).start()

    def wait(kb, sem):
        pltpu.make_async_copy(k_hbm.at[idx_v.at[0]], kb, sem).wait()

    def qk_chunk(kb, c, m_vec):
        def lsub_body(j, m):
            lb = j * 8
            rows = [jnp.full((16,), lb + l, jnp.int32) for l in range(8)]
            acc = [None] * 8
            ci = hlane
            for tb in range(4):
                qv = [q_r[tb * 16 + t, :] for t in range(16)]
                for t in range(16):
                    flat = base64 + ci
                    s2 = flat >> 7
                    l2 = flat & 127
                    for l in range(8):
                        kv = plsc.load_gather(kb, [rows[l], s2, l2])
                        tmp = kv * qv[t]
                        acc[l] = tmp if tb == 0 and t == 0 else acc[l] + tmp
                    if tb < 3 or t < 15:
                        ci = (ci + 1) & 63
            for l in range(8):
                logits[c * _CH + lb + l, :] = acc[l]
                m = jnp.maximum(m, acc[l])
            return m
        return lax.fori_loop(0, _CH // 8, lsub_body, m_vec)

    fire(0, kb0, sem0)
    fire(1, kb1, sem1)

    def pair1(i, m):
        c = i * 2
        wait(kb0, sem0)
        m = qk_chunk(kb0, c, m)
        fire(c + 2, kb0, sem0)
        wait(kb1, sem1)
        m = qk_chunk(kb1, c + 1, m)
        fire(c + 3, kb1, sem1)
        return m

    m_vec = lax.fori_loop(0, _NCH // 2, pair1,
                          jnp.full((16,), -3e38, jnp.float32))

    mbuf[...] = m_vec
    pltpu.sync_copy(logits, lg_hbm.at[b])
    pltpu.sync_copy(mbuf, m_hbm.at[b])


@functools.partial(
    pl.kernel,
    out_type=(
        jax.ShapeDtypeStruct((_B, _H, _LV), jnp.float32),
        jax.ShapeDtypeStruct((_B, _H), jnp.float32),
    ),
    mesh=_mesh,
    scratch_types=(
        pltpu.VMEM((_NCH, _CH), jnp.int32),
        pltpu.VMEM((_CH, 8, 128), jnp.float32),
        pltpu.VMEM((_CH, 8, 128), jnp.float32),
        pltpu.VMEM((_L, _H), jnp.float32),
        pltpu.VMEM((_H,), jnp.float32),
        pltpu.VMEM((_LV, _H), jnp.float32),
        pltpu.VMEM((_H, _LV), jnp.float32),
        pltpu.VMEM((_H,), jnp.float32),
        pltpu.SemaphoreType.DMA,
        pltpu.SemaphoreType.DMA,
    ),
    compiler_params=_params,
)
def _sc_pv(v_hbm, idx_hbm, lg_hbm, m_hbm, out_hbm, lse_hbm,
           idx_v, kb0, kb1, logits, mbuf, accV, out_buf, lse_buf,
           sem0, sem1):
    b = lax.axis_index("s") * 2 + lax.axis_index("c")

    pltpu.sync_copy(idx_hbm.at[b], idx_v)
    pltpu.sync_copy(lg_hbm.at[b], logits)
    pltpu.sync_copy(m_hbm.at[b], mbuf)
    m_vec = mbuf[...]

    hlane = lax.broadcasted_iota(jnp.int32, (16,), 0)
    base64 = hlane * _D

    def fire(c, kb, sem):
        @pl.when(c < _NCH)
        def _():
            pltpu.make_async_copy(v_hbm.at[idx_v.at[c]], kb, sem).start()

    def wait(kb, sem):
        pltpu.make_async_copy(v_hbm.at[idx_v.at[0]], kb, sem).wait()

    def pv_chunk(kb, c, ssum):
        def lsub_body(j, s):
            lb = j * 8
            rows = [jnp.full((16,), lb + l, jnp.int32) for l in range(8)]
            ps = []
            for l in range(8):
                p = jnp.exp(logits[c * _CH + lb + l, :] - m_vec)
                s = s + p
                ps.append(p)
            for q in range(4):
                acc = [accV[q * 16 + t, :] for t in range(16)]
                ci0 = hlane + (q * 16)
                for l in range(8):
                    ci = ci0
                    tt = []
                    for t in range(16):
                        flat = base64 + ci
                        tt.append(plsc.load_gather(
                            kb, [rows[l], flat >> 7, flat & 127]))
                        if t < 15:
                            ci = (ci + 1) & 63
                    for t in range(16):
                        acc[t] = acc[t] + ps[l] * tt[t]
                for t in range(16):
                    accV[q * 16 + t, :] = acc[t]
            return s
        return lax.fori_loop(0, _CH // 8, lsub_body, ssum)

    fire(0, kb0, sem0)
    fire(1, kb1, sem1)

    zero = jnp.zeros((16,), jnp.float32)
    for v in range(_LV):
        accV[v, :] = zero

    def pair2(i, s):
        c = i * 2
        wait(kb0, sem0)
        s = pv_chunk(kb0, c, s)
        fire(c + 2, kb0, sem0)
        wait(kb1, sem1)
        s = pv_chunk(kb1, c + 1, s)
        fire(c + 3, kb1, sem1)
        return s

    ssum = lax.fori_loop(0, _NCH // 2, pair2, zero)

    rec = 1.0 / ssum
    for t in range(_LV):
        accV[t, :] = accV[t, :] * rec

    vi0 = lax.broadcasted_iota(jnp.int32, (16,), 0)
    for h in range(_H):
        hr = jnp.full((16,), h, jnp.int32)
        for vb in range(4):
            tidx = (vi0 + (vb * 16 - h + 64)) & 63
            out_buf[h, pl.ds(vb * 16, 16)] = plsc.load_gather(
                accV, [tidx, hr])

    bits = plsc.bitcast(ssum, jnp.int32)
    ex = (bits >> 23) - 127
    mant = plsc.bitcast((bits & 0x7FFFFF) | 0x3F800000, jnp.float32)
    y = ex.astype(jnp.float32) * _LN2 + (mant - 1.0) * _LN2 + 0.0298
    y = y + ssum * jnp.exp(-y) - 1.0
    y = y + ssum * jnp.exp(-y) - 1.0
    lse_buf[...] = m_vec + y

    pltpu.sync_copy(out_buf, out_hbm.at[b])
    pltpu.sync_copy(lse_buf, lse_hbm.at[b])


def kernel(q, k_buffer, v_buffer, kv_indptr, kv_indices, num_kv_splits):
    qt = (q * _SCALE).transpose(0, 2, 1)
    rot = (jnp.arange(_D)[:, None] + jnp.arange(_H)[None, :]) % _D
    q_rot = jnp.take_along_axis(qt, rot[None, :, :], axis=1)
    idx3 = kv_indices.reshape(_B, _NCH, _CH)
    k3 = k_buffer.reshape(_T, 8, 128)
    v3 = v_buffer.reshape(_T, 8, 128)
    lg, m = _sc_qk(q_rot, k3, idx3)
    out, lse = _sc_pv(v3, idx3, lg, m)
    return out[:, :, None, :], lse[:, :, None]


# speedup vs baseline: 1.0579x; 1.0579x over previous
import functools

import jax
import jax.numpy as jnp
from jax import lax
from jax.experimental import pallas as pl
from jax.experimental.pallas import tpu as pltpu
from jax.experimental.pallas import tpu_sc as plsc

_B, _H, _D, _LV = 32, 16, 64, 64
_T = 65536
_L = _T // _B
_CH = 32
_NCH = _L // _CH
_SCALE = 0.125
_LN2 = 0.6931471805599453

_mesh = plsc.VectorSubcoreMesh(core_axis_name="c", subcore_axis_name="s")


@functools.partial(
    pl.kernel,
    out_type=(
        jax.ShapeDtypeStruct((_B, _H, _LV), jnp.float32),
        jax.ShapeDtypeStruct((_B, _H), jnp.float32),
    ),
    mesh=_mesh,
    scratch_types=(
        pltpu.VMEM((_NCH, _CH), jnp.int32),
        pltpu.VMEM((_D, _H), jnp.float32),
        pltpu.VMEM((_CH, 8, 128), jnp.float32),
        pltpu.VMEM((_CH, 8, 128), jnp.float32),
        pltpu.VMEM((_L, _H), jnp.float32),
        pltpu.VMEM((_LV, _H), jnp.float32),
        pltpu.VMEM((_H, _LV), jnp.float32),
        pltpu.VMEM((_H,), jnp.float32),
        pltpu.SemaphoreType.DMA,
        pltpu.SemaphoreType.DMA,
    ),
    compiler_params=pltpu.CompilerParams(use_tc_tiling_on_sc=False,
                                         needs_layout_passes=False,
                                         disable_bounds_checks=True),
)
def _sc_attn(qr_hbm, k_hbm, v_hbm, idx_hbm, out_hbm, lse_hbm,
             idx_v, q_r, kb0, kb1, logits, accV, out_buf, lse_buf,
             sem0, sem1):
    b = lax.axis_index("s") * 2 + lax.axis_index("c")

    pltpu.sync_copy(idx_hbm.at[b], idx_v)
    pltpu.sync_copy(qr_hbm.at[b], q_r)

    hlane = lax.broadcasted_iota(jnp.int32, (16,), 0)
    base64 = hlane * _D

    def fire(src_hbm, c, kb, sem):
        @pl.when(c < _NCH)
        def _():
            pltpu.make_async_copy(src_hbm.at[idx_v.at[c]], kb, sem).start()

    def wait(src_hbm, kb, sem):
        pltpu.make_async_copy(src_hbm.at[idx_v.at[0]], kb, sem).wait()

    def qk_chunk(kb, c, m_vec):
        def lsub_body(j, m):
            lb = j * 8
            rows = [jnp.full((16,), lb + l, jnp.int32) for l in range(8)]
            acc = [None] * 8
            ci = hlane
            for tb in range(4):
                qv = [q_r[tb * 16 + t, :] for t in range(16)]
                for t in range(16):
                    flat = base64 + ci
                    s2 = flat >> 7
                    l2 = flat & 127
                    for l in range(8):
                        kv = plsc.load_gather(kb, [rows[l], s2, l2])
                        tmp = kv * qv[t]
                        acc[l] = tmp if tb == 0 and t == 0 else acc[l] + tmp
                    if tb < 3 or t < 15:
                        ci = (ci + 1) & 63
            for l in range(8):
                logits[c * _CH + lb + l, :] = acc[l]
                m = jnp.maximum(m, acc[l])
            return m
        return lax.fori_loop(0, _CH // 8, lsub_body, m_vec)

    def pv_chunk(kb, c, m_vec, ssum):
        def lsub_body(j, s):
            lb = j * 8
            rows = [jnp.full((16,), lb + l, jnp.int32) for l in range(8)]
            ps = []
            for l in range(8):
                p = jnp.exp(logits[c * _CH + lb + l, :] - m_vec)
                s = s + p
                ps.append(p)
            for q in range(4):
                acc = [accV[q * 16 + t, :] for t in range(16)]
                ci0 = hlane + (q * 16)
                for l in range(8):
                    ci = ci0
                    tt = []
                    for t in range(16):
                        flat = base64 + ci
                        tt.append(plsc.load_gather(
                            kb, [rows[l], flat >> 7, flat & 127]))
                        if t < 15:
                            ci = (ci + 1) & 63
                    for t in range(16):
                        acc[t] = acc[t] + ps[l] * tt[t]
                for t in range(16):
                    accV[q * 16 + t, :] = acc[t]
            return s
        return lax.fori_loop(0, _CH // 8, lsub_body, ssum)

    fire(k_hbm, 0, kb0, sem0)
    fire(k_hbm, 1, kb1, sem1)

    def pair1(i, m):
        c = i * 2
        wait(k_hbm, kb0, sem0)
        m = qk_chunk(kb0, c, m)
        fire(k_hbm, c + 2, kb0, sem0)
        wait(k_hbm, kb1, sem1)
        m = qk_chunk(kb1, c + 1, m)
        fire(k_hbm, c + 3, kb1, sem1)
        return m

    m_vec = lax.fori_loop(0, _NCH // 2, pair1,
                          jnp.full((16,), -3e38, jnp.float32))

    fire(v_hbm, 0, kb0, sem0)
    fire(v_hbm, 1, kb1, sem1)

    zero = jnp.zeros((16,), jnp.float32)
    for v in range(_LV):
        accV[v, :] = zero

    def pair2(i, s):
        c = i * 2
        wait(v_hbm, kb0, sem0)
        s = pv_chunk(kb0, c, m_vec, s)
        fire(v_hbm, c + 2, kb0, sem0)
        wait(v_hbm, kb1, sem1)
        s = pv_chunk(kb1, c + 1, m_vec, s)
        fire(v_hbm, c + 3, kb1, sem1)
        return s

    ssum = lax.fori_loop(0, _NCH // 2, pair2, zero)

    rec = 1.0 / ssum
    for t in range(_LV):
        accV[t, :] = accV[t, :] * rec

    vi0 = lax.broadcasted_iota(jnp.int32, (16,), 0)
    for h in range(_H):
        hr = jnp.full((16,), h, jnp.int32)
        for vb in range(4):
            tidx = (vi0 + (vb * 16 - h + 64)) & 63
            out_buf[h, pl.ds(vb * 16, 16)] = plsc.load_gather(
                accV, [tidx, hr])

    bits = plsc.bitcast(ssum, jnp.int32)
    ex = (bits >> 23) - 127
    mant = plsc.bitcast((bits & 0x7FFFFF) | 0x3F800000, jnp.float32)
    y = ex.astype(jnp.float32) * _LN2 + (mant - 1.0) * _LN2 + 0.0298
    y = y + ssum * jnp.exp(-y) - 1.0
    y = y + ssum * jnp.exp(-y) - 1.0
    lse_buf[...] = m_vec + y

    pltpu.sync_copy(out_buf, out_hbm.at[b])
    pltpu.sync_copy(lse_buf, lse_hbm.at[b])


def kernel(q, k_buffer, v_buffer, kv_indptr, kv_indices, num_kv_splits):
    qt = (q * _SCALE).transpose(0, 2, 1)
    rot = (jnp.arange(_D)[:, None] + jnp.arange(_H)[None, :]) % _D
    q_rot = jnp.take_along_axis(qt, rot[None, :, :], axis=1)
    idx3 = kv_indices.reshape(_B, _NCH, _CH)
    k3 = k_buffer.reshape(_T, 8, 128)
    v3 = v_buffer.reshape(_T, 8, 128)
    out, lse = _sc_attn(q_rot, k3, v3, idx3)
    return out[:, :, None, :], lse[:, :, None]
